# baseline (speedup 1.0000x reference)
import functools

import jax
import jax.numpy as jnp
from jax import lax
from jax.experimental import pallas as pl
from jax.experimental.pallas import tpu as pltpu
from jax.experimental.pallas import tpu_sc as plsc

_N = 4096
_D = 16
_L = 16
_NC = 2
_NS = 16
_NW = _NC * _NS
_PERW = _N // _NS
_CPAD = 128
_CPW = _CPAD // _NW

def _rsqrt16(x):
    i = lax.bitcast_convert_type(x, jnp.int32)
    y = lax.bitcast_convert_type(jnp.int32(0x5F3759DF) - (i >> 1), jnp.float32)
    for _ in range(4):
        y = y * (1.5 - 0.5 * x * y * y)
    return y


def _extract(v, lo, lanes):
    return jnp.sum(jnp.where(lanes == lo, v, jnp.zeros((_L,), v.dtype)))


def _body(emb_hbm, lab_hbm, part_hbm, nu_hbm, esort_hbm,
          emb_l, lab_l, hist_l, allh_l, counts_l, offs_l, curs_l, pos_l,
          es_l, acc_l, nu_l, hist_sh):
    c = lax.axis_index("c")
    s = lax.axis_index("s")
    w = c * _NS + s
    lanes = lax.iota(jnp.int32, _L)
    _I0 = jnp.zeros((_L,), jnp.int32)

    pltpu.sync_copy(lab_hbm, lab_l)
    base = s * _PERW
    pltpu.sync_copy(emb_hbm.at[pl.ds(base, _PERW)], emb_l)

    def norm_body(t, _):
        v = emb_l[t, :]
        ss = jnp.sum(v * v)
        r = _rsqrt16(jnp.maximum(jnp.full((_L,), ss), 1e-24))
        emb_l[t, :] = v * r
        return 0
    lax.fori_loop(0, _PERW, norm_body, 0)

    for ci in range(_CPAD // _L):
        hist_l[pl.ds(ci * _L, _L)] = _I0

    def hist_blk(tb, _):
        lv = lab_l[pl.ds(base + tb * _L, _L)]
        for j in range(_L):
            lbl = lv[j]
            ch = (lbl >> 4) * _L
            hit = lanes == (lbl & (_L - 1))
            hist_l[pl.ds(ch, _L)] = (hist_l[pl.ds(ch, _L)]
                                     + hit.astype(jnp.int32))
        return 0
    lax.fori_loop(0, _PERW // _L, hist_blk, 0)

    pltpu.sync_copy(hist_l, hist_sh.at[s])
    plsc.subcore_barrier()
    pltpu.sync_copy(hist_sh, allh_l)

    for ci in range(_CPAD // _L):
        sl = pl.ds(ci * _L, _L)
        tot = _I0
        pre = _I0
        sv = jnp.full((_L,), s)
        for r in range(_NS):
            row = allh_l[r, sl]
            pre = pre + jnp.where(jnp.full((_L,), r) < sv, row, _I0)
            tot = tot + row
        counts_l[sl] = tot
        curs_l[sl] = pre

    run = jnp.int32(0)
    nun = jnp.int32(0)
    for ci in range(_CPAD // _L):
        sl = pl.ds(ci * _L, _L)
        c16 = counts_l[sl]
        cs = plsc.cumsum(c16)
        excl = cs - c16 + run
        offs_l[sl] = excl
        curs_l[sl] = curs_l[sl] + excl
        run = run + cs[_L - 1]
        nun = nun + jnp.sum((c16 > 0).astype(jnp.int32))
    nu_l[...] = jnp.full((_L,), nun).astype(jnp.float32)

    def pos_blk(tb, _):
        lv = lab_l[pl.ds(base + tb * _L, _L)]
        posv = _I0
        for j in range(_L):
            lbl = lv[j]
            ch = (lbl >> 4) * _L
            hit = lanes == (lbl & (_L - 1))
            v = curs_l[pl.ds(ch, _L)]
            p0 = _extract(v, lbl & (_L - 1), lanes)
            curs_l[pl.ds(ch, _L)] = v + hit.astype(jnp.int32)
            posv = jnp.where(lanes == j, jnp.full((_L,), p0), posv)
        pos_l[tb >> 3, pl.ds((tb & 7) * _L, _L)] = posv
        return 0
    lax.fori_loop(0, _PERW // _L, pos_blk, 0)

    for ch in range(_PERW // 128):
        pltpu.sync_copy(emb_l.at[pl.ds(ch * 128, 128)],
                        esort_hbm.at[pos_l.at[ch]])
    plsc.subcore_barrier()

    pltpu.sync_copy(esort_hbm, es_l)

    def cluster(l_idx, acc_v):
        ch = (l_idx >> 4) * _L
        lo = l_idx & (_L - 1)
        s0 = _extract(offs_l[pl.ds(ch, _L)], lo, lanes)
        n = _extract(counts_l[pl.ds(ch, _L)], lo, lanes)
        nb = (n + _L - 1) >> 4

        def i_body(ii, acc_v):
            p = s0 + ii
            av = es_l[p, :]
            a = [av[k] for k in range(_D)]
            jb0 = ii >> 4

            def j_body(jb, acc_v):
                jbase = s0 + jb * _L
                rows = jnp.minimum(jbase + lanes, _N - 1)
                sq = jnp.zeros((_L,), jnp.float32)
                for k in range(_D):
                    b = plsc.load_gather(
                        es_l, [rows, jnp.full((_L,), k, jnp.int32)])
                    d = b - a[k]
                    sq = sq + d * d
                jl = jb * _L + lanes
                valid = (jl > ii) & (jl < n)
                dist = sq * _rsqrt16(jnp.maximum(sq, 1e-30))
                return acc_v + jnp.where(valid, dist, 0.0)

            return lax.fori_loop(jb0, nb, j_body, acc_v)

        return lax.fori_loop(0, n, i_body, acc_v)

    acc = jnp.zeros((_L,), jnp.float32)
    for m in range(_CPW):
        acc = cluster(w + m * _NW, acc)

    acc_l[...] = acc + acc
    pltpu.sync_copy(acc_l, part_hbm.at[w])

    @pl.when(w == 0)
    def _():
        pltpu.sync_copy(nu_l, nu_hbm)


def kernel(embeddings, cluster_labels):
    labels = cluster_labels.astype(jnp.int32)
    mesh = plsc.VectorSubcoreMesh(core_axis_name="c", subcore_axis_name="s",
                                  num_cores=_NC, num_subcores=_NS)
    fn = pl.kernel(
        _body,
        out_type=[
            jax.ShapeDtypeStruct((_NW, _L), jnp.float32),
            jax.ShapeDtypeStruct((_L,), jnp.float32),
            jax.ShapeDtypeStruct((_N, _D), jnp.float32),
        ],
        mesh=mesh,
        compiler_params=pltpu.CompilerParams(needs_layout_passes=False,
                                             use_tc_tiling_on_sc=False),
        scratch_types=[
            pltpu.VMEM((_PERW, _D), jnp.float32),
            pltpu.VMEM((_N,), jnp.int32),
            pltpu.VMEM((_CPAD,), jnp.int32),
            pltpu.VMEM((_NS, _CPAD), jnp.int32),
            pltpu.VMEM((_CPAD,), jnp.int32),
            pltpu.VMEM((_CPAD,), jnp.int32),
            pltpu.VMEM((_CPAD,), jnp.int32),
            pltpu.VMEM((_PERW // 128, 128), jnp.int32),
            pltpu.VMEM((_N, _D), jnp.float32),
            pltpu.VMEM((_L,), jnp.float32),
            pltpu.VMEM((_L,), jnp.float32),
            pltpu.VMEM_SHARED((_NS, _CPAD), jnp.int32),
        ],
    )
    part, nu, _ = fn(embeddings, labels)
    return jnp.sum(part) / nu[0]


# speedup vs baseline: 1.0483x; 1.0483x over previous
import functools

import jax
import jax.numpy as jnp
from jax import lax
from jax.experimental import pallas as pl
from jax.experimental.pallas import tpu as pltpu
from jax.experimental.pallas import tpu_sc as plsc

_N = 4096
_D = 16
_L = 16
_NC = 2
_NS = 16
_NW = _NC * _NS
_CPAD = 128
_CPW = _CPAD // _NW
_CAP = _N + _L
_NBLK = _N // _L


def _rsqrt16(x):
    i = lax.bitcast_convert_type(x, jnp.int32)
    y = lax.bitcast_convert_type(jnp.int32(0x5F3759DF) - (i >> 1), jnp.float32)
    for _ in range(3):
        y = y * (1.5 - 0.5 * x * y * y)
    return y


def _body(emb_hbm, lab_hbm, part_hbm, nu_hbm,
          es_l, lab_l, memb_l, rn_l, acc_l, nu_l):
    c = lax.axis_index("c")
    s = lax.axis_index("s")
    w = c * _NS + s
    lanes = lax.iota(jnp.int32, _L)
    f0 = jnp.zeros((_L,), jnp.float32)

    pltpu.sync_copy(lab_hbm, lab_l)
    pltpu.sync_copy(emb_hbm, es_l)

    def scan_blk(tb, curs):
        lv = lab_l[pl.ds(tb * _L, _L)]
        idxv = tb * _L + lanes
        new = []
        for m in range(_CPW):
            hit = lv == (w + m * _NW)
            plsc.store_compressed(memb_l.at[m, pl.ds(curs[m], _L)], idxv,
                                  mask=hit)
            new.append(curs[m]
                       + plsc.all_reduce_population_count(hit)[0])
        return tuple(new)
    cnts = lax.fori_loop(0, _NBLK, scan_blk,
                         tuple(jnp.int32(0) for _ in range(_CPW)))
    for m in range(_CPW):
        memb_l[m, pl.ds(cnts[m], _L)] = jnp.zeros((_L,), jnp.int32)

    def rn_cluster(m, cnt):
        nb = (cnt + _L - 1) >> 4

        def rn_blk(b, _):
            iv = memb_l[m, pl.ds(b * _L, _L)]
            ssv = f0
            for j in range(_L):
                row = es_l[iv[j], :]
                ssj = jnp.sum(row * row)
                ssv = jnp.where(lanes == j, jnp.full((_L,), ssj), ssv)
            rn_l[m, pl.ds(b * _L, _L)] = _rsqrt16(jnp.maximum(ssv, 1e-24))
            return 0
        lax.fori_loop(0, nb, rn_blk, 0)

    for m in range(_CPW):
        rn_cluster(m, cnts[m])

    def pair_cluster(m, n, acc_v):
        nb = (n + _L - 1) >> 4

        def i_body(ii, acc_v):
            ib = ii >> 4
            a_idx_v = memb_l[m, pl.ds(ib * _L, _L)]
            a_idx = jnp.sum(jnp.where(lanes == (ii & (_L - 1)), a_idx_v,
                                      jnp.zeros((_L,), jnp.int32)))
            rn_a_v = rn_l[m, pl.ds(ib * _L, _L)]
            rn_a = jnp.sum(jnp.where(lanes == (ii & (_L - 1)), rn_a_v, f0))
            av = es_l[a_idx, :]
            a = [av[k] for k in range(_D)]

            def j_body(jb, acc_v):
                rows = memb_l[m, pl.ds(jb * _L, _L)]
                d0 = f0
                d1 = f0
                for k in range(0, _D, 2):
                    b0 = plsc.load_gather(
                        es_l, [rows, jnp.full((_L,), k, jnp.int32)])
                    b1 = plsc.load_gather(
                        es_l, [rows, jnp.full((_L,), k + 1, jnp.int32)])
                    d0 = d0 + b0 * a[k]
                    d1 = d1 + b1 * a[k + 1]
                rnv = rn_l[m, pl.ds(jb * _L, _L)]
                sq = 2.0 - (2.0 * rn_a) * ((d0 + d1) * rnv)
                sq = jnp.maximum(sq, 1e-30)
                jl = jb * _L + lanes
                valid = (jl > ii) & (jl < n)
                dist = sq * _rsqrt16(sq)
                return acc_v + jnp.where(valid, dist, 0.0)

            return lax.fori_loop(ib, nb, j_body, acc_v)

        return lax.fori_loop(0, n, i_body, acc_v)

    acc = f0
    nun = jnp.int32(0)
    for m in range(_CPW):
        acc = pair_cluster(m, cnts[m], acc)
        nun = nun + jnp.where(cnts[m] > 0, 1, 0)

    acc_l[...] = acc + acc
    nu_l[...] = jnp.where(lanes == 0, jnp.full((_L,), nun), 0
                          ).astype(jnp.float32)
    pltpu.sync_copy(acc_l, part_hbm.at[w])
    pltpu.sync_copy(nu_l, nu_hbm.at[w])


def kernel(embeddings, cluster_labels):
    labels = cluster_labels.astype(jnp.int32)
    mesh = plsc.VectorSubcoreMesh(core_axis_name="c", subcore_axis_name="s",
                                  num_cores=_NC, num_subcores=_NS)
    fn = pl.kernel(
        _body,
        out_type=[
            jax.ShapeDtypeStruct((_NW, _L), jnp.float32),
            jax.ShapeDtypeStruct((_NW, _L), jnp.float32),
        ],
        mesh=mesh,
        compiler_params=pltpu.CompilerParams(needs_layout_passes=False,
                                             use_tc_tiling_on_sc=False),
        scratch_types=[
            pltpu.VMEM((_N, _D), jnp.float32),
            pltpu.VMEM((_N,), jnp.int32),
            pltpu.VMEM((_CPW, _CAP), jnp.int32),
            pltpu.VMEM((_CPW, _CAP), jnp.float32),
            pltpu.VMEM((_L,), jnp.float32),
            pltpu.VMEM((_L,), jnp.float32),
        ],
    )
    part, nu = fn(embeddings, labels)
    return jnp.sum(part) / jnp.sum(nu)
